# Initial kernel scaffold; baseline (speedup 1.0000x reference)
import jax
import jax.numpy as jnp
from jax.experimental import pallas as pl


def kernel(x, in_proj_w, conv_w, conv_b, x_proj_w, dt_proj_w, dt_proj_b, A_log, D, out_proj_w, ln_g, ln_b):
    raise NotImplementedError("write your pallas kernel here")


# speedup vs baseline: 17.7301x; 17.7301x over previous
import jax
import jax.numpy as jnp
from jax import lax
from jax.experimental import pallas as pl
from jax.experimental.pallas import tpu as pltpu

DM = 768
DI = 1536
NS = 16
DTR = 48
DCONV = 4
NLAYER = 4
NB = 2
L = 1024
EPS = 1e-6
CT = 32
NC = L // CT
FPAD = 128


def _silu(v):
    return v * (1.0 / (1.0 + jnp.exp(-v)))


def _mamba_body(x_ref, wu_ref, wz_ref, cw_ref, cb_ref, wx_ref, wdt_ref,
                dtb_ref, alog_ref, d_ref, wo_ref, lng_ref, lnb_ref,
                out_ref,
                u_s, u2_s, xdbl_s, dA_s, hch_s, c3_s):
    dep = pl.program_id(1)

    @pl.when(dep == 0)
    def _():
        out_ref[0] = x_ref[0]

    xin = out_ref[0]
    xin_b = xin.astype(jnp.bfloat16)

    u_s[:] = jnp.dot(xin_b, wu_ref[0], preferred_element_type=jnp.float32)

    cw = cw_ref[0]
    u = u_s[:]
    acc = u * cw[3:4, :] + cb_ref[0]
    for k in range(1, DCONV):
        sh = jnp.concatenate(
            [jnp.zeros((k, DI), jnp.float32), u[:L - k, :]], axis=0)
        acc = acc + sh * cw[3 - k:4 - k, :]
    u2 = _silu(acc)
    u2_s[:] = u2

    xdbl = jnp.dot(u2.astype(jnp.bfloat16), wx_ref[0],
                   preferred_element_type=jnp.float32)
    xdbl_s[:] = xdbl.reshape(L, 1, FPAD)

    dpre = jnp.dot(xdbl.astype(jnp.bfloat16), wdt_ref[0],
                   preferred_element_type=jnp.float32) + dtb_ref[0]
    delta = jnp.maximum(dpre, 0.0) + jnp.log(1.0 + jnp.exp(-jnp.abs(dpre)))
    u_s[:] = delta

    a_t = -jnp.exp(alog_ref[0])

    def chunk_body(c, h):
        t0 = c * CT
        delta_c = u_s[pl.ds(t0, CT), :].reshape(CT, 1, DI)
        dA_s[:] = jnp.exp(delta_c * a_t[None, :, :])
        du_c = delta_c * u2_s[pl.ds(t0, CT), :].reshape(CT, 1, DI)
        b3 = jnp.swapaxes(xdbl_s[pl.ds(t0, CT), :, DTR:DTR + NS], 1, 2)
        c3_s[:] = jnp.swapaxes(
            xdbl_s[pl.ds(t0, CT), :, DTR + NS:DTR + 2 * NS], 1, 2)
        hch_s[:] = du_c * b3

        def step(t, hc):
            hn = dA_s[t] * hc + hch_s[t]
            hch_s[t] = hn
            return hn

        h = lax.fori_loop(0, CT, step, h)
        y_c = jnp.sum(hch_s[:] * c3_s[:], axis=1)
        u_s[pl.ds(t0, CT), :] = y_c
        return h

    h0 = jnp.zeros((NS, DI), jnp.float32)
    lax.fori_loop(0, NC, chunk_body, h0)

    y = u_s[:] + u2_s[:] * d_ref[0]
    z = jnp.dot(xin_b, wz_ref[0], preferred_element_type=jnp.float32)
    g = y * _silu(z)
    o = jnp.dot(g.astype(jnp.bfloat16), wo_ref[0],
                preferred_element_type=jnp.float32) + xin
    mu = jnp.mean(o, axis=-1, keepdims=True)
    d0 = o - mu
    var = jnp.mean(d0 * d0, axis=-1, keepdims=True)
    out_ref[0] = d0 * lax.rsqrt(var + EPS) * lng_ref[0] + lnb_ref[0]


def kernel(x, in_proj_w, conv_w, conv_b, x_proj_w, dt_proj_w, dt_proj_b,
           A_log, D, out_proj_w, ln_g, ln_b):
    wu = jnp.swapaxes(in_proj_w[:, :DI, :], 1, 2).astype(jnp.bfloat16)
    wz = jnp.swapaxes(in_proj_w[:, DI:, :], 1, 2).astype(jnp.bfloat16)
    cw = jnp.pad(jnp.swapaxes(conv_w, 1, 2), ((0, 0), (0, 8 - DCONV), (0, 0)))
    cb = conv_b[:, None, :]
    wx = jnp.pad(jnp.swapaxes(x_proj_w, 1, 2),
                 ((0, 0), (0, 0), (0, FPAD - (DTR + 2 * NS)))
                 ).astype(jnp.bfloat16)
    wdt = jnp.pad(jnp.swapaxes(dt_proj_w, 1, 2),
                  ((0, 0), (0, FPAD - DTR), (0, 0))).astype(jnp.bfloat16)
    dtb = dt_proj_b[:, None, :]
    alog = jnp.swapaxes(A_log, 1, 2)
    dd = D[:, None, :]
    wo = jnp.swapaxes(out_proj_w, 1, 2).astype(jnp.bfloat16)
    lng = ln_g[:, None, :]
    lnb = ln_b[:, None, :]

    def bmap(b, l):
        return (b, 0, 0)

    def lmap(b, l):
        return (l, 0, 0)

    return pl.pallas_call(
        _mamba_body,
        grid=(NB, NLAYER),
        in_specs=[
            pl.BlockSpec((1, L, DM), bmap),
            pl.BlockSpec((1, DM, DI), lmap),
            pl.BlockSpec((1, DM, DI), lmap),
            pl.BlockSpec((1, 8, DI), lmap),
            pl.BlockSpec((1, 1, DI), lmap),
            pl.BlockSpec((1, DI, FPAD), lmap),
            pl.BlockSpec((1, FPAD, DI), lmap),
            pl.BlockSpec((1, 1, DI), lmap),
            pl.BlockSpec((1, NS, DI), lmap),
            pl.BlockSpec((1, 1, DI), lmap),
            pl.BlockSpec((1, DI, DM), lmap),
            pl.BlockSpec((1, 1, DM), lmap),
            pl.BlockSpec((1, 1, DM), lmap),
        ],
        out_specs=pl.BlockSpec((1, L, DM), bmap),
        out_shape=jax.ShapeDtypeStruct((NB, L, DM), jnp.float32),
        scratch_shapes=[
            pltpu.VMEM((L, DI), jnp.float32),
            pltpu.VMEM((L, DI), jnp.float32),
            pltpu.VMEM((L, 1, FPAD), jnp.float32),
            pltpu.VMEM((CT, NS, DI), jnp.float32),
            pltpu.VMEM((CT, NS, DI), jnp.float32),
            pltpu.VMEM((CT, NS, 1), jnp.float32),
        ],
        compiler_params=pltpu.CompilerParams(
            dimension_semantics=("parallel", "arbitrary"),
            vmem_limit_bytes=56 * 1024 * 1024,
        ),
    )(x, wu, wz, cw, cb, wx, wdt, dtb, alog, dd, wo, lng, lnb)


# speedup vs baseline: 21.4933x; 1.2122x over previous
import jax
import jax.numpy as jnp
from jax import lax
from jax.experimental import pallas as pl
from jax.experimental.pallas import tpu as pltpu

DM = 768
DI = 1536
NS = 16
DTR = 48
DCONV = 4
NLAYER = 4
NB = 2
L = 1024
EPS = 1e-6
CT = 32
NC = L // CT
FPAD = 128


def _silu(v):
    return v * (1.0 / (1.0 + jnp.exp(-v)))


def _mamba_body(x_ref, wu_ref, wz_ref, cw_ref, cb_ref, wx_ref, wdt_ref,
                dtb_ref, alog_ref, d_ref, wo_ref, lng_ref, lnb_ref,
                out_ref,
                u_s, u2_s, xdbl_s, dA_s, hch_s):
    dep = pl.program_id(1)

    @pl.when(dep == 0)
    def _():
        out_ref[0] = x_ref[0]

    xin = out_ref[0]
    xin_b = xin.astype(jnp.bfloat16)

    u_s[:] = jnp.dot(xin_b, wu_ref[0], preferred_element_type=jnp.float32)

    cw = cw_ref[0]
    u = u_s[:]
    acc = u * cw[3:4, :] + cb_ref[0]
    for k in range(1, DCONV):
        sh = jnp.concatenate(
            [jnp.zeros((k, DI), jnp.float32), u[:L - k, :]], axis=0)
        acc = acc + sh * cw[3 - k:4 - k, :]
    u2 = _silu(acc)
    u2_s[:] = u2

    xdbl = jnp.dot(u2.astype(jnp.bfloat16), wx_ref[0],
                   preferred_element_type=jnp.float32)
    xdbl_s[:] = xdbl

    dpre = jnp.dot(xdbl.astype(jnp.bfloat16), wdt_ref[0],
                   preferred_element_type=jnp.float32) + dtb_ref[0]
    delta = jnp.maximum(dpre, 0.0) + jnp.log(1.0 + jnp.exp(-jnp.abs(dpre)))
    u_s[:] = delta

    a2 = (-jnp.exp(alog_ref[0])) * jnp.float32(1.4426950408889634)
    a_rep = pltpu.repeat(a2, CT, axis=0)
    BD = jnp.where(
        lax.broadcasted_iota(jnp.int32, (CT, CT * NS), 1) // NS
        == lax.broadcasted_iota(jnp.int32, (CT, CT * NS), 0),
        1.0, 0.0).astype(jnp.float32)

    def rep_lanes(m16):
        m128 = jnp.concatenate([m16] * 8, axis=1)
        return pltpu.repeat(m128, CT * NS // 128, axis=1)

    def chunk_body(c, h):
        t0 = c * CT
        delta_c = u_s[pl.ds(t0, CT), :]
        delta3 = jnp.repeat(delta_c, NS, axis=0)
        dA_s[:] = jnp.exp2(delta3 * a_rep)
        du = delta_c * u2_s[pl.ds(t0, CT), :]
        sb = rep_lanes(xdbl_s[pl.ds(t0, CT), DTR:DTR + NS]) * BD
        hch_s[:] = jnp.dot(jnp.swapaxes(sb, 0, 1), du,
                           preferred_element_type=jnp.float32)

        def step(t, hc):
            r0 = t * NS
            hn = dA_s[pl.ds(r0, NS), :] * hc + hch_s[pl.ds(r0, NS), :]
            hch_s[pl.ds(r0, NS), :] = hn
            return hn

        h = lax.fori_loop(0, CT, step, h)
        sc = rep_lanes(xdbl_s[pl.ds(t0, CT), DTR + NS:DTR + 2 * NS]) * BD
        y_c = jnp.dot(sc, hch_s[:], preferred_element_type=jnp.float32)
        u_s[pl.ds(t0, CT), :] = y_c
        return h

    h0 = jnp.zeros((NS, DI), jnp.float32)
    lax.fori_loop(0, NC, chunk_body, h0)

    y = u_s[:] + u2_s[:] * d_ref[0]
    z = jnp.dot(xin_b, wz_ref[0], preferred_element_type=jnp.float32)
    g = y * _silu(z)
    o = jnp.dot(g.astype(jnp.bfloat16), wo_ref[0],
                preferred_element_type=jnp.float32) + xin
    mu = jnp.mean(o, axis=-1, keepdims=True)
    d0 = o - mu
    var = jnp.mean(d0 * d0, axis=-1, keepdims=True)
    out_ref[0] = d0 * lax.rsqrt(var + EPS) * lng_ref[0] + lnb_ref[0]


def kernel(x, in_proj_w, conv_w, conv_b, x_proj_w, dt_proj_w, dt_proj_b,
           A_log, D, out_proj_w, ln_g, ln_b):
    wu = jnp.swapaxes(in_proj_w[:, :DI, :], 1, 2).astype(jnp.bfloat16)
    wz = jnp.swapaxes(in_proj_w[:, DI:, :], 1, 2).astype(jnp.bfloat16)
    cw = jnp.pad(jnp.swapaxes(conv_w, 1, 2), ((0, 0), (0, 8 - DCONV), (0, 0)))
    cb = conv_b[:, None, :]
    wx = jnp.pad(jnp.swapaxes(x_proj_w, 1, 2),
                 ((0, 0), (0, 0), (0, FPAD - (DTR + 2 * NS)))
                 ).astype(jnp.bfloat16)
    wdt = jnp.pad(jnp.swapaxes(dt_proj_w, 1, 2),
                  ((0, 0), (0, FPAD - DTR), (0, 0))).astype(jnp.bfloat16)
    dtb = dt_proj_b[:, None, :]
    alog = jnp.swapaxes(A_log, 1, 2)
    dd = D[:, None, :]
    wo = jnp.swapaxes(out_proj_w, 1, 2).astype(jnp.bfloat16)
    lng = ln_g[:, None, :]
    lnb = ln_b[:, None, :]

    def bmap(b, l):
        return (b, 0, 0)

    def lmap(b, l):
        return (l, 0, 0)

    return pl.pallas_call(
        _mamba_body,
        grid=(NB, NLAYER),
        in_specs=[
            pl.BlockSpec((1, L, DM), bmap),
            pl.BlockSpec((1, DM, DI), lmap),
            pl.BlockSpec((1, DM, DI), lmap),
            pl.BlockSpec((1, 8, DI), lmap),
            pl.BlockSpec((1, 1, DI), lmap),
            pl.BlockSpec((1, DI, FPAD), lmap),
            pl.BlockSpec((1, FPAD, DI), lmap),
            pl.BlockSpec((1, 1, DI), lmap),
            pl.BlockSpec((1, NS, DI), lmap),
            pl.BlockSpec((1, 1, DI), lmap),
            pl.BlockSpec((1, DI, DM), lmap),
            pl.BlockSpec((1, 1, DM), lmap),
            pl.BlockSpec((1, 1, DM), lmap),
        ],
        out_specs=pl.BlockSpec((1, L, DM), bmap),
        out_shape=jax.ShapeDtypeStruct((NB, L, DM), jnp.float32),
        scratch_shapes=[
            pltpu.VMEM((L, DI), jnp.float32),
            pltpu.VMEM((L, DI), jnp.float32),
            pltpu.VMEM((L, FPAD), jnp.float32),
            pltpu.VMEM((CT * NS, DI), jnp.float32),
            pltpu.VMEM((CT * NS, DI), jnp.float32),
        ],
        compiler_params=pltpu.CompilerParams(
            dimension_semantics=("arbitrary", "arbitrary"),
            vmem_limit_bytes=56 * 1024 * 1024,
        ),
    )(x, wu, wz, cw, cb, wx, wdt, dtb, alog, dd, wo, lng, lnb)


# speedup vs baseline: 22.4906x; 1.0464x over previous
import jax
import jax.numpy as jnp
from jax import lax
from jax.experimental import pallas as pl
from jax.experimental.pallas import tpu as pltpu

DM = 768
DI = 1536
NS = 16
DTR = 48
DCONV = 4
NLAYER = 4
NB = 2
L = 1024
EPS = 1e-6
CT = 32
NC = L // CT
FPAD = 128


LOG2E = 1.4426950408889634


def _silu(v):
    return v * (1.0 / (1.0 + jnp.exp2(v * jnp.float32(-LOG2E))))


def _mamba_body(x_ref, wu_ref, wz_ref, cw_ref, cb_ref, wx_ref, wdt_ref,
                dtb_ref, alog_ref, d_ref, wo_ref, lng_ref, lnb_ref,
                out_ref,
                uc_s, u2_s, xdbl_s, dA_s, hch_s):
    dep = pl.program_id(1)

    @pl.when(dep == 0)
    def _():
        out_ref[0] = x_ref[0]

    xin = out_ref[0]
    xin_b = xin.astype(jnp.bfloat16)

    uc_s[0:8, :] = jnp.zeros((8, DI), jnp.float32)
    uc_s[pl.ds(8, L), :] = jnp.dot(xin_b, wu_ref[0],
                                   preferred_element_type=jnp.float32)

    cw = cw_ref[0]
    acc = uc_s[pl.ds(8, L), :] * cw[3:4, :] + cb_ref[0]
    for k in range(1, DCONV):
        acc = acc + uc_s[pl.ds(8 - k, L), :] * cw[3 - k:4 - k, :]
    u2 = _silu(acc)
    u2_s[:] = u2

    xdbl = jnp.dot(u2.astype(jnp.bfloat16), wx_ref[0],
                   preferred_element_type=jnp.float32)
    xdbl_s[:] = xdbl

    dpre = jnp.dot(xdbl.astype(jnp.bfloat16), wdt_ref[0],
                   preferred_element_type=jnp.float32) + dtb_ref[0]
    delta = jnp.maximum(dpre, 0.0) + jnp.log(1.0 + jnp.exp(-jnp.abs(dpre)))
    uc_s[pl.ds(0, L), :] = delta

    a2 = (-jnp.exp(alog_ref[0])) * jnp.float32(1.4426950408889634)
    a_rep = pltpu.repeat(a2, CT, axis=0)
    BD = jnp.where(
        lax.broadcasted_iota(jnp.int32, (CT, CT * NS), 1) // NS
        == lax.broadcasted_iota(jnp.int32, (CT, CT * NS), 0),
        1.0, 0.0).astype(jnp.float32)

    def rep_lanes(m16):
        m128 = jnp.concatenate([m16] * 8, axis=1)
        return pltpu.repeat(m128, CT * NS // 128, axis=1)

    def chunk_body(c, h):
        t0 = c * CT
        delta_c = uc_s[pl.ds(t0, CT), :]
        delta3 = jnp.repeat(delta_c, NS, axis=0)
        dA_s[:] = jnp.exp2(delta3 * a_rep)
        du = delta_c * u2_s[pl.ds(t0, CT), :]
        sb = rep_lanes(xdbl_s[pl.ds(t0, CT), DTR:DTR + NS]) * BD
        hch_s[:] = jnp.dot(jnp.swapaxes(sb, 0, 1), du,
                           preferred_element_type=jnp.float32)

        def step(t, hc):
            r0 = t * NS
            hn = dA_s[pl.ds(r0, NS), :] * hc + hch_s[pl.ds(r0, NS), :]
            hch_s[pl.ds(r0, NS), :] = hn
            return hn

        h = lax.fori_loop(0, CT, step, h, unroll=2)
        sc = rep_lanes(xdbl_s[pl.ds(t0, CT), DTR + NS:DTR + 2 * NS]) * BD
        y_c = jnp.dot(sc, hch_s[:], preferred_element_type=jnp.float32)
        uc_s[pl.ds(t0, CT), :] = y_c
        return h

    h0 = jnp.zeros((NS, DI), jnp.float32)
    lax.fori_loop(0, NC, chunk_body, h0)

    y = uc_s[pl.ds(0, L), :] + u2_s[:] * d_ref[0]
    z = jnp.dot(xin_b, wz_ref[0], preferred_element_type=jnp.float32)
    g = y * _silu(z)
    o = jnp.dot(g.astype(jnp.bfloat16), wo_ref[0],
                preferred_element_type=jnp.float32) + xin
    mu = jnp.mean(o, axis=-1, keepdims=True)
    d0 = o - mu
    var = jnp.mean(d0 * d0, axis=-1, keepdims=True)
    out_ref[0] = d0 * lax.rsqrt(var + EPS) * lng_ref[0] + lnb_ref[0]


def kernel(x, in_proj_w, conv_w, conv_b, x_proj_w, dt_proj_w, dt_proj_b,
           A_log, D, out_proj_w, ln_g, ln_b):
    wu = jnp.swapaxes(in_proj_w[:, :DI, :], 1, 2).astype(jnp.bfloat16)
    wz = jnp.swapaxes(in_proj_w[:, DI:, :], 1, 2).astype(jnp.bfloat16)
    cw = jnp.pad(jnp.swapaxes(conv_w, 1, 2), ((0, 0), (0, 8 - DCONV), (0, 0)))
    cb = conv_b[:, None, :]
    wx = jnp.pad(jnp.swapaxes(x_proj_w, 1, 2),
                 ((0, 0), (0, 0), (0, FPAD - (DTR + 2 * NS)))
                 ).astype(jnp.bfloat16)
    wdt = jnp.pad(jnp.swapaxes(dt_proj_w, 1, 2),
                  ((0, 0), (0, FPAD - DTR), (0, 0))).astype(jnp.bfloat16)
    dtb = dt_proj_b[:, None, :]
    alog = jnp.swapaxes(A_log, 1, 2)
    dd = D[:, None, :]
    wo = jnp.swapaxes(out_proj_w, 1, 2).astype(jnp.bfloat16)
    lng = ln_g[:, None, :]
    lnb = ln_b[:, None, :]

    def bmap(b, l):
        return (b, 0, 0)

    def lmap(b, l):
        return (l, 0, 0)

    return pl.pallas_call(
        _mamba_body,
        grid=(NB, NLAYER),
        in_specs=[
            pl.BlockSpec((1, L, DM), bmap),
            pl.BlockSpec((1, DM, DI), lmap),
            pl.BlockSpec((1, DM, DI), lmap),
            pl.BlockSpec((1, 8, DI), lmap),
            pl.BlockSpec((1, 1, DI), lmap),
            pl.BlockSpec((1, DI, FPAD), lmap),
            pl.BlockSpec((1, FPAD, DI), lmap),
            pl.BlockSpec((1, 1, DI), lmap),
            pl.BlockSpec((1, NS, DI), lmap),
            pl.BlockSpec((1, 1, DI), lmap),
            pl.BlockSpec((1, DI, DM), lmap),
            pl.BlockSpec((1, 1, DM), lmap),
            pl.BlockSpec((1, 1, DM), lmap),
        ],
        out_specs=pl.BlockSpec((1, L, DM), bmap),
        out_shape=jax.ShapeDtypeStruct((NB, L, DM), jnp.float32),
        scratch_shapes=[
            pltpu.VMEM((L + 8, DI), jnp.float32),
            pltpu.VMEM((L, DI), jnp.float32),
            pltpu.VMEM((L, FPAD), jnp.float32),
            pltpu.VMEM((CT * NS, DI), jnp.float32),
            pltpu.VMEM((CT * NS, DI), jnp.float32),
        ],
        compiler_params=pltpu.CompilerParams(
            dimension_semantics=("arbitrary", "arbitrary"),
            vmem_limit_bytes=60000 * 1024,
        ),
    )(x, wu, wz, cw, cb, wx, wdt, dtb, alog, dd, wo, lng, lnb)


# speedup vs baseline: 25.9193x; 1.1525x over previous
import jax
import jax.numpy as jnp
from jax import lax
from jax.experimental import pallas as pl
from jax.experimental.pallas import tpu as pltpu

DM = 768
DI = 1536
NS = 16
DTR = 48
DCONV = 4
NLAYER = 4
NB = 2
L = 1024
EPS = 1e-6
CT = 16
NC = L // CT
FPAD = 128


LOG2E = 1.4426950408889634


def _silu(v):
    return v * (1.0 / (1.0 + jnp.exp2(v * jnp.float32(-LOG2E))))


def _mamba_body(x_ref, wu_ref, wz_ref, cw_ref, cb_ref, wx_ref, wdt_ref,
                dtb_ref, alog_ref, d_ref, wo_ref, lng_ref, lnb_ref,
                out_ref,
                uc_s, u2_s, xdbl_s, dA0_s, hch0_s, dA1_s, hch1_s):
    dep = pl.program_id(1)

    @pl.when(dep == 0)
    def _():
        out_ref[0] = x_ref[0]

    xin = out_ref[0]
    xin_b = xin.astype(jnp.bfloat16)

    uc_s[0:8, :] = jnp.zeros((8, DI), jnp.float32)
    uc_s[pl.ds(8, L), :] = jnp.dot(xin_b, wu_ref[0],
                                   preferred_element_type=jnp.float32)

    cw = cw_ref[0]
    acc = uc_s[pl.ds(8, L), :] * cw[3:4, :] + cb_ref[0]
    for k in range(1, DCONV):
        acc = acc + uc_s[pl.ds(8 - k, L), :] * cw[3 - k:4 - k, :]
    u2 = _silu(acc)
    u2_s[:] = u2

    xdbl = jnp.dot(u2.astype(jnp.bfloat16), wx_ref[0],
                   preferred_element_type=jnp.float32)
    xdbl_s[:] = xdbl

    dpre = jnp.dot(xdbl.astype(jnp.bfloat16), wdt_ref[0],
                   preferred_element_type=jnp.float32) + dtb_ref[0]
    delta = jnp.maximum(dpre, 0.0) + jnp.log(1.0 + jnp.exp(-jnp.abs(dpre)))
    uc_s[pl.ds(0, L), :] = delta

    a2 = (-jnp.exp(alog_ref[0])) * jnp.float32(1.4426950408889634)
    a_rep = pltpu.repeat(a2, CT, axis=0)
    BD = jnp.where(
        lax.broadcasted_iota(jnp.int32, (CT, CT * NS), 1) // NS
        == lax.broadcasted_iota(jnp.int32, (CT, CT * NS), 0),
        1.0, 0.0).astype(jnp.float32)

    def rep_lanes(m16):
        m128 = jnp.concatenate([m16] * 8, axis=1)
        return pltpu.repeat(m128, CT * NS // 128, axis=1)

    def chunk_build(c, dA_b, hch_b):
        t0 = c * CT
        delta_c = uc_s[pl.ds(t0, CT), :]
        delta3 = jnp.repeat(delta_c, NS, axis=0)
        dA_b[:] = jnp.exp2(delta3 * a_rep)
        du = delta_c * u2_s[pl.ds(t0, CT), :]
        sb = rep_lanes(xdbl_s[pl.ds(t0, CT), DTR:DTR + NS]) * BD
        hch_b[:] = jnp.dot(jnp.swapaxes(sb, 0, 1), du,
                           preferred_element_type=jnp.float32)

    def chunk_scan(c, h, dA_b, hch_b):
        for t in range(CT):
            r0 = t * NS
            h = dA_b[r0:r0 + NS, :] * h + hch_b[r0:r0 + NS, :]
            hch_b[r0:r0 + NS, :] = h
        t0 = c * CT
        sc = rep_lanes(xdbl_s[pl.ds(t0, CT), DTR + NS:DTR + 2 * NS]) * BD
        y_c = jnp.dot(sc, hch_b[:], preferred_element_type=jnp.float32)
        uc_s[pl.ds(t0, CT), :] = y_c
        return h

    chunk_build(0, dA0_s, hch0_s)

    def pair_body(k, h):
        c0 = 2 * k
        chunk_build(c0 + 1, dA1_s, hch1_s)
        h = chunk_scan(c0, h, dA0_s, hch0_s)
        chunk_build(jnp.where(c0 + 2 < NC, c0 + 2, 0), dA0_s, hch0_s)
        h = chunk_scan(c0 + 1, h, dA1_s, hch1_s)
        return h

    h0 = jnp.zeros((NS, DI), jnp.float32)
    lax.fori_loop(0, NC // 2, pair_body, h0)

    y = uc_s[pl.ds(0, L), :] + u2_s[:] * d_ref[0]
    z = jnp.dot(xin_b, wz_ref[0], preferred_element_type=jnp.float32)
    g = y * _silu(z)
    o = jnp.dot(g.astype(jnp.bfloat16), wo_ref[0],
                preferred_element_type=jnp.float32) + xin
    mu = jnp.mean(o, axis=-1, keepdims=True)
    d0 = o - mu
    var = jnp.mean(d0 * d0, axis=-1, keepdims=True)
    out_ref[0] = d0 * lax.rsqrt(var + EPS) * lng_ref[0] + lnb_ref[0]


def kernel(x, in_proj_w, conv_w, conv_b, x_proj_w, dt_proj_w, dt_proj_b,
           A_log, D, out_proj_w, ln_g, ln_b):
    wu = jnp.swapaxes(in_proj_w[:, :DI, :], 1, 2).astype(jnp.bfloat16)
    wz = jnp.swapaxes(in_proj_w[:, DI:, :], 1, 2).astype(jnp.bfloat16)
    cw = jnp.pad(jnp.swapaxes(conv_w, 1, 2), ((0, 0), (0, 8 - DCONV), (0, 0)))
    cb = conv_b[:, None, :]
    wx = jnp.pad(jnp.swapaxes(x_proj_w, 1, 2),
                 ((0, 0), (0, 0), (0, FPAD - (DTR + 2 * NS)))
                 ).astype(jnp.bfloat16)
    wdt = jnp.pad(jnp.swapaxes(dt_proj_w, 1, 2),
                  ((0, 0), (0, FPAD - DTR), (0, 0))).astype(jnp.bfloat16)
    dtb = dt_proj_b[:, None, :]
    alog = jnp.swapaxes(A_log, 1, 2)
    dd = D[:, None, :]
    wo = jnp.swapaxes(out_proj_w, 1, 2).astype(jnp.bfloat16)
    lng = ln_g[:, None, :]
    lnb = ln_b[:, None, :]

    def bmap(b, l):
        return (b, 0, 0)

    def lmap(b, l):
        return (l, 0, 0)

    return pl.pallas_call(
        _mamba_body,
        grid=(NB, NLAYER),
        in_specs=[
            pl.BlockSpec((1, L, DM), bmap),
            pl.BlockSpec((1, DM, DI), lmap),
            pl.BlockSpec((1, DM, DI), lmap),
            pl.BlockSpec((1, 8, DI), lmap),
            pl.BlockSpec((1, 1, DI), lmap),
            pl.BlockSpec((1, DI, FPAD), lmap),
            pl.BlockSpec((1, FPAD, DI), lmap),
            pl.BlockSpec((1, 1, DI), lmap),
            pl.BlockSpec((1, NS, DI), lmap),
            pl.BlockSpec((1, 1, DI), lmap),
            pl.BlockSpec((1, DI, DM), lmap),
            pl.BlockSpec((1, 1, DM), lmap),
            pl.BlockSpec((1, 1, DM), lmap),
        ],
        out_specs=pl.BlockSpec((1, L, DM), bmap),
        out_shape=jax.ShapeDtypeStruct((NB, L, DM), jnp.float32),
        scratch_shapes=[
            pltpu.VMEM((L + 8, DI), jnp.float32),
            pltpu.VMEM((L, DI), jnp.float32),
            pltpu.VMEM((L, FPAD), jnp.float32),
            pltpu.VMEM((CT * NS, DI), jnp.float32),
            pltpu.VMEM((CT * NS, DI), jnp.float32),
            pltpu.VMEM((CT * NS, DI), jnp.float32),
            pltpu.VMEM((CT * NS, DI), jnp.float32),
        ],
        compiler_params=pltpu.CompilerParams(
            dimension_semantics=("arbitrary", "arbitrary"),
            vmem_limit_bytes=60000 * 1024,
        ),
    )(x, wu, wz, cw, cb, wx, wdt, dtb, alog, dd, wo, lng, lnb)
